# baseline (speedup 1.0000x reference)
import functools

import jax
import jax.numpy as jnp
from jax import lax
from jax.experimental import pallas as pl
from jax.experimental.pallas import tpu as pltpu
from jax.experimental.pallas import tpu_sc as plsc

N = 10000
D = 128
E = 320000
G = 128
C = 10
EPS = 1e-5

NC = 2
NS = 16
NW = NC * NS
B = 80
ITERS = 128
HALF = ITERS // 2
EPAD = NW * ITERS * B
NPAD = 10240
ZROWS = NPAD // NS

_mesh = plsc.VectorSubcoreMesh(core_axis_name="c", subcore_axis_name="s")

f32 = jnp.float32


DW = 128


@functools.partial(
    pl.kernel,
    out_type=jax.ShapeDtypeStruct((NC, NPAD, DW), f32),
    mesh=_mesh,
    scratch_types=[
        pltpu.VMEM((ITERS, B), jnp.int32),
        pltpu.VMEM((B, DW), f32),
        pltpu.VMEM_SHARED((NPAD, DW), f32),
        [pltpu.SemaphoreType.DMA] * 4,
    ],
)
def _deg_kernel(dst_hbm, zcol_hbm, ones_hbm, out_hbm, idx_v, ones_v, sh,
                sems):
    cid = lax.axis_index("c")
    sid = lax.axis_index("s")
    wid = sid * NC + cid
    seg = NPAD // NS
    pltpu.sync_copy(zcol_hbm, sh.at[pl.ds(sid * seg, seg), :])
    pltpu.sync_copy(ones_hbm, ones_v)
    pltpu.sync_copy(dst_hbm.at[wid], idx_v)
    plsc.subcore_barrier()

    def scat(c, s):
        pltpu.async_copy(ones_v, sh.at[idx_v.at[c]], sems[s], add=True)

    def wait(s):
        pltpu.make_async_copy(ones_v, sh.at[idx_v.at[0]], sems[s]).wait()

    def body(k, carry):
        for b in range(4):
            pl.when(k > 0)(functools.partial(wait, b))
            scat(4 * k + b, b)
        return carry

    lax.fori_loop(0, ITERS // 4, body, 0, unroll=False)
    for s in range(4):
        wait(s)
    plsc.subcore_barrier()
    pltpu.sync_copy(sh.at[pl.ds(sid * seg, seg), :],
                    out_hbm.at[cid, pl.ds(sid * seg, seg), :])


@functools.partial(
    pl.kernel,
    out_type=jax.ShapeDtypeStruct((NC, NPAD, D), f32),
    mesh=_mesh,
    scratch_types=[
        pltpu.VMEM((HALF, B), jnp.int32),
        pltpu.VMEM((HALF, B), jnp.int32),
        [pltpu.VMEM((B, D), f32)] * 2,
        pltpu.VMEM_SHARED((NPAD, D), f32),
        [pltpu.SemaphoreType.DMA] * 2,
        [pltpu.SemaphoreType.DMA] * 2,
    ],
)
def _scatter_kernel(g_hbm, srcs_hbm, dsts_hbm, zrow_hbm, out_hbm,
                    idxs_v, idxd_v, rows, sh, gsems, ssems):
    cid = lax.axis_index("c")
    sid = lax.axis_index("s")
    wid = sid * NC + cid
    pltpu.sync_copy(zrow_hbm, sh.at[pl.ds(sid * ZROWS, ZROWS), :])
    plsc.subcore_barrier()

    def gath(c, b):
        pltpu.async_copy(g_hbm.at[idxs_v.at[c]], rows[b], gsems[b])

    def wait_g(b):
        pltpu.make_async_copy(g_hbm.at[idxs_v.at[0]], rows[b],
                              gsems[b]).wait()

    def scat(c, b):
        pltpu.async_copy(rows[b], sh.at[idxd_v.at[c]], ssems[b], add=True)

    def wait_s(b):
        pltpu.make_async_copy(rows[b], sh.at[idxd_v.at[0]], ssems[b]).wait()

    def body(k, carry):
        pl.when(k > 0)(functools.partial(wait_s, 0))
        gath(2 * k, 0)

        def lag0():
            wait_g(1)
            scat(2 * k - 1, 1)

        pl.when(k > 0)(lag0)
        pl.when(k > 0)(functools.partial(wait_s, 1))
        gath(2 * k + 1, 1)
        wait_g(0)
        scat(2 * k, 0)
        return carry

    for p in range(2):
        pltpu.sync_copy(srcs_hbm.at[wid * 2 + p], idxs_v)
        pltpu.sync_copy(dsts_hbm.at[wid * 2 + p], idxd_v)
        lax.fori_loop(0, HALF // 2, body, 0, unroll=False)
        wait_g(1)
        scat(HALF - 1, 1)
        wait_s(0)
        wait_s(1)
    plsc.subcore_barrier()
    pltpu.sync_copy(sh.at[pl.ds(sid * ZROWS, ZROWS), :],
                    out_hbm.at[cid, pl.ds(sid * ZROWS, ZROWS), :])


def _pre_body(x_ref, we_ref, be_ref, w0_ref, degp_ref, g0_ref, dinv_ref):
    degp = degp_ref[...]
    deg = degp[0, :N, 0:1] + degp[1, :N, 0:1] + 1.0
    dinv = lax.rsqrt(deg)
    h0 = jnp.dot(x_ref[...], we_ref[...],
                 preferred_element_type=f32,
                 precision=lax.Precision.HIGHEST) + be_ref[...][None, :]
    g0_ref[...] = jnp.dot(h0, w0_ref[...],
                          preferred_element_type=f32,
                          precision=lax.Precision.HIGHEST) * dinv
    dinv_ref[...] = dinv


_pre_call = pl.pallas_call(
    _pre_body,
    out_shape=(jax.ShapeDtypeStruct((N, D), f32),
               jax.ShapeDtypeStruct((N, 1), f32)),
)


def _post_body(acc_ref, g_ref, dinv_ref, cb_ref, bg_ref, bb_ref, wn_ref,
               gn_ref):
    acc = acc_ref[...]
    dinv = dinv_ref[...]
    s = acc[0, :N, :] + acc[1, :N, :] + g_ref[...]
    t = dinv * s + cb_ref[...][None, :]
    scale = (bg_ref[...] * (1.0 / jnp.sqrt(1.0 + EPS)))[None, :]
    h = jnp.maximum(t * scale + bb_ref[...][None, :], 0.0)
    gn_ref[...] = jnp.dot(h, wn_ref[...],
                          preferred_element_type=f32,
                          precision=lax.Precision.HIGHEST) * dinv


_post_call = pl.pallas_call(
    _post_body,
    out_shape=jax.ShapeDtypeStruct((N, D), f32),
)


def _final_body(acc_ref, g_ref, dinv_ref, cb_ref, bg_ref, bb_ref, batch_ref,
                m1w_ref, m1b_ref, mbg_ref, mbb_ref, m2w_ref, m2b_ref,
                b2g_ref, b2b_ref, ow_ref, ob_ref, out_ref):
    bnscale = 1.0 / jnp.sqrt(1.0 + EPS)
    acc = acc_ref[...]
    dinv = dinv_ref[...]
    s = acc[0, :N, :] + acc[1, :N, :] + g_ref[...]
    t = dinv * s + cb_ref[...][None, :]
    h = jnp.maximum(t * (bg_ref[...] * bnscale)[None, :] + bb_ref[...][None, :],
                    0.0)
    batch = batch_ref[...]
    onehot = (batch[None, :] ==
              lax.broadcasted_iota(jnp.int32, (G, N), 0)).astype(f32)
    cnt = jnp.sum(onehot, axis=1, keepdims=True)
    pooled = jnp.dot(onehot, h, preferred_element_type=f32,
                     precision=lax.Precision.HIGHEST) / jnp.maximum(cnt, 1.0)
    z = jnp.dot(pooled, m1w_ref[...], preferred_element_type=f32,
                precision=lax.Precision.HIGHEST) + m1b_ref[...][None, :]
    z = z * (mbg_ref[...] * bnscale)[None, :] + mbb_ref[...][None, :]
    z = jnp.maximum(z, 0.0)
    z = jnp.dot(z, m2w_ref[...], preferred_element_type=f32,
                precision=lax.Precision.HIGHEST) + m2b_ref[...][None, :]
    z = z * (b2g_ref[...] * bnscale)[None, :] + b2b_ref[...][None, :]
    out_ref[...] = jnp.dot(z, ow_ref[...], preferred_element_type=f32,
                           precision=lax.Precision.HIGHEST) + ob_ref[...][None, :]


_final_call = pl.pallas_call(
    _final_body,
    out_shape=jax.ShapeDtypeStruct((G, C), f32),
)


def kernel(x, edge_index, batch, W_enc, b_enc, conv_W, conv_b, bn_g, bn_b,
           mlp1_W, mlp1_b, mlp_bn_g, mlp_bn_b, mlp2_W, mlp2_b, bn2_g, bn2_b,
           out_W, out_b):
    ei = edge_index.astype(jnp.int32)
    pad = EPAD - E
    src3 = jnp.concatenate(
        [ei[0], jnp.zeros((pad,), jnp.int32)]).reshape(NW, ITERS, B)
    dst3 = jnp.concatenate(
        [ei[1], jnp.full((pad,), NPAD - 1, jnp.int32)]).reshape(NW, ITERS, B)
    src2 = src3.reshape(NW * 2, HALF, B)
    dst2 = dst3.reshape(NW * 2, HALF, B)
    zcol = jnp.zeros((NPAD // NS, DW), f32)
    ones_chunk = jnp.ones((B, DW), f32)
    zrow = jnp.zeros((ZROWS, D), f32)

    degp = _deg_kernel(dst3, zcol, ones_chunk)
    g0, dinv = _pre_call(x, W_enc, b_enc, conv_W[0], degp)
    acc0 = _scatter_kernel(g0, src2, dst2, zrow)
    g1 = _post_call(acc0, g0, dinv, conv_b[0], bn_g[0], bn_b[0], conv_W[1])
    acc1 = _scatter_kernel(g1, src2, dst2, zrow)
    g2 = _post_call(acc1, g1, dinv, conv_b[1], bn_g[1], bn_b[1], conv_W[2])
    acc2 = _scatter_kernel(g2, src2, dst2, zrow)
    return _final_call(acc2, g2, dinv, conv_b[2], bn_g[2], bn_b[2], batch,
                       mlp1_W, mlp1_b, mlp_bn_g, mlp_bn_b, mlp2_W, mlp2_b,
                       bn2_g, bn2_b, out_W, out_b)


# speedup vs baseline: 3.1454x; 3.1454x over previous
import functools

import jax
import jax.numpy as jnp
from jax import lax
from jax.experimental import pallas as pl
from jax.experimental.pallas import tpu as pltpu
from jax.experimental.pallas import tpu_sc as plsc

N = 10000
D = 128
E = 320000
G = 128
C = 10
EPS = 1e-5

NC = 2
NS = 16
NW = NC * NS
B = 80
ITERS = 128
HALF = ITERS // 2
EPAD = NW * ITERS * B
NPAD = 10240
ZROWS = NPAD // NS

_mesh = plsc.VectorSubcoreMesh(core_axis_name="c", subcore_axis_name="s")

f32 = jnp.float32


DW = 128


@functools.partial(
    pl.kernel,
    out_type=jax.ShapeDtypeStruct((NC, NPAD, DW), f32),
    mesh=_mesh,
    scratch_types=[
        pltpu.VMEM((ITERS, B), jnp.int32),
        pltpu.VMEM((B, DW), f32),
        pltpu.VMEM_SHARED((NPAD, DW), f32),
        [pltpu.SemaphoreType.DMA] * 4,
    ],
)
def _deg_kernel(dst_hbm, zcol_hbm, ones_hbm, out_hbm, idx_v, ones_v, sh,
                sems):
    cid = lax.axis_index("c")
    sid = lax.axis_index("s")
    wid = sid * NC + cid
    seg = NPAD // NS
    pltpu.sync_copy(zcol_hbm, sh.at[pl.ds(sid * seg, seg), :])
    pltpu.sync_copy(ones_hbm, ones_v)
    pltpu.sync_copy(dst_hbm.at[wid], idx_v)
    plsc.subcore_barrier()

    def scat(c, s):
        pltpu.async_copy(ones_v, sh.at[idx_v.at[c]], sems[s], add=True)

    def wait(s):
        pltpu.make_async_copy(ones_v, sh.at[idx_v.at[0]], sems[s]).wait()

    def body(k, carry):
        for b in range(4):
            pl.when(k > 0)(functools.partial(wait, b))
            scat(4 * k + b, b)
        return carry

    lax.fori_loop(0, ITERS // 4, body, 0, unroll=False)
    for s in range(4):
        wait(s)
    plsc.subcore_barrier()
    pltpu.sync_copy(sh.at[pl.ds(sid * seg, seg), :],
                    out_hbm.at[cid, pl.ds(sid * seg, seg), :])


@functools.partial(
    pl.kernel,
    out_type=jax.ShapeDtypeStruct((NC, NPAD, D), f32),
    mesh=_mesh,
    scratch_types=[
        pltpu.VMEM((HALF, B), jnp.int32),
        pltpu.VMEM((HALF, B), jnp.int32),
        [pltpu.VMEM((B, D), f32)] * 2,
        pltpu.VMEM_SHARED((NPAD, D), f32),
        [pltpu.SemaphoreType.DMA] * 2,
        [pltpu.SemaphoreType.DMA] * 2,
    ],
)
def _scatter_kernel(g_hbm, srcs_hbm, dsts_hbm, zrow_hbm, out_hbm,
                    idxs_v, idxd_v, rows, sh, gsems, ssems):
    cid = lax.axis_index("c")
    sid = lax.axis_index("s")
    wid = sid * NC + cid
    pltpu.sync_copy(zrow_hbm, sh.at[pl.ds(sid * ZROWS, ZROWS), :])
    plsc.subcore_barrier()

    def gath(c, b):
        pltpu.async_copy(g_hbm.at[idxs_v.at[c]], rows[b], gsems[b])

    def wait_g(b):
        pltpu.make_async_copy(g_hbm.at[idxs_v.at[0]], rows[b],
                              gsems[b]).wait()

    def scat(c, b):
        pltpu.async_copy(rows[b], sh.at[idxd_v.at[c]], ssems[b], add=True)

    def wait_s(b):
        pltpu.make_async_copy(rows[b], sh.at[idxd_v.at[0]], ssems[b]).wait()

    def body(k, carry):
        pl.when(k > 0)(functools.partial(wait_s, 0))
        gath(2 * k, 0)

        def lag0():
            wait_g(1)
            scat(2 * k - 1, 1)

        pl.when(k > 0)(lag0)
        pl.when(k > 0)(functools.partial(wait_s, 1))
        gath(2 * k + 1, 1)
        wait_g(0)
        scat(2 * k, 0)
        return carry

    for p in range(2):
        pltpu.sync_copy(srcs_hbm.at[wid * 2 + p], idxs_v)
        pltpu.sync_copy(dsts_hbm.at[wid * 2 + p], idxd_v)
        lax.fori_loop(0, HALF // 2, body, 0, unroll=False)
        wait_g(1)
        scat(HALF - 1, 1)
        wait_s(0)
        wait_s(1)
    plsc.subcore_barrier()
    pltpu.sync_copy(sh.at[pl.ds(sid * ZROWS, ZROWS), :],
                    out_hbm.at[cid, pl.ds(sid * ZROWS, ZROWS), :])


def _pre_body(x_ref, we_ref, be_ref, w0_ref, degp_ref, g0_ref, dinv_ref):
    degp = degp_ref[...]
    deg = degp[0, :N, 0:1] + degp[1, :N, 0:1] + 1.0
    dinv = lax.rsqrt(deg)
    h0 = jnp.dot(x_ref[...], we_ref[...],
                 preferred_element_type=f32,
                 precision=lax.Precision.HIGHEST) + be_ref[...][None, :]
    g0_ref[...] = jnp.dot(h0, w0_ref[...],
                          preferred_element_type=f32,
                          precision=lax.Precision.HIGHEST) * dinv
    dinv_ref[...] = dinv


_pre_call = pl.pallas_call(
    _pre_body,
    out_shape=(jax.ShapeDtypeStruct((N, D), f32),
               jax.ShapeDtypeStruct((N, 1), f32)),
)


def _post_body(acc_ref, g_ref, dinv_ref, cb_ref, bg_ref, bb_ref, wn_ref,
               gn_ref):
    acc = acc_ref[...]
    dinv = dinv_ref[...]
    s = acc[0, :N, :] + acc[1, :N, :] + g_ref[...]
    t = dinv * s + cb_ref[...][None, :]
    scale = (bg_ref[...] * (1.0 / jnp.sqrt(1.0 + EPS)))[None, :]
    h = jnp.maximum(t * scale + bb_ref[...][None, :], 0.0)
    gn_ref[...] = jnp.dot(h, wn_ref[...],
                          preferred_element_type=f32,
                          precision=lax.Precision.HIGHEST) * dinv


_post_call = pl.pallas_call(
    _post_body,
    out_shape=jax.ShapeDtypeStruct((N, D), f32),
)


def _final_body(acc_ref, g_ref, dinv_ref, cb_ref, bg_ref, bb_ref, batch_ref,
                m1w_ref, m1b_ref, mbg_ref, mbb_ref, m2w_ref, m2b_ref,
                b2g_ref, b2b_ref, ow_ref, ob_ref, out_ref):
    bnscale = 1.0 / jnp.sqrt(1.0 + EPS)
    acc = acc_ref[...]
    dinv = dinv_ref[...]
    s = acc[0, :N, :] + acc[1, :N, :] + g_ref[...]
    t = dinv * s + cb_ref[...][None, :]
    h = jnp.maximum(t * (bg_ref[...] * bnscale)[None, :] + bb_ref[...][None, :],
                    0.0)
    batch = batch_ref[...]
    onehot = (batch[None, :] ==
              lax.broadcasted_iota(jnp.int32, (G, N), 0)).astype(f32)
    cnt = jnp.sum(onehot, axis=1, keepdims=True)
    pooled = jnp.dot(onehot, h, preferred_element_type=f32,
                     precision=lax.Precision.HIGHEST) / jnp.maximum(cnt, 1.0)
    z = jnp.dot(pooled, m1w_ref[...], preferred_element_type=f32,
                precision=lax.Precision.HIGHEST) + m1b_ref[...][None, :]
    z = z * (mbg_ref[...] * bnscale)[None, :] + mbb_ref[...][None, :]
    z = jnp.maximum(z, 0.0)
    z = jnp.dot(z, m2w_ref[...], preferred_element_type=f32,
                precision=lax.Precision.HIGHEST) + m2b_ref[...][None, :]
    z = z * (b2g_ref[...] * bnscale)[None, :] + b2b_ref[...][None, :]
    out_ref[...] = jnp.dot(z, ow_ref[...], preferred_element_type=f32,
                           precision=lax.Precision.HIGHEST) + ob_ref[...][None, :]


_final_call = pl.pallas_call(
    _final_body,
    out_shape=jax.ShapeDtypeStruct((G, C), f32),
)


def kernel(x, edge_index, batch, W_enc, b_enc, conv_W, conv_b, bn_g, bn_b,
           mlp1_W, mlp1_b, mlp_bn_g, mlp_bn_b, mlp2_W, mlp2_b, bn2_g, bn2_b,
           out_W, out_b):
    ei = edge_index.astype(jnp.int32)
    pad = EPAD - E
    pad_ar = jnp.arange(pad, dtype=jnp.int32)
    src3 = jnp.concatenate(
        [ei[0], pad_ar % N]).reshape(NW, ITERS, B)
    dst3 = jnp.concatenate(
        [ei[1], N + pad_ar % (NPAD - N)]).reshape(NW, ITERS, B)
    src2 = src3.reshape(NW * 2, HALF, B)
    dst2 = dst3.reshape(NW * 2, HALF, B)
    zcol = jnp.zeros((NPAD // NS, DW), f32)
    ones_chunk = jnp.ones((B, DW), f32)
    zrow = jnp.zeros((ZROWS, D), f32)

    degp = _deg_kernel(dst3, zcol, ones_chunk)
    g0, dinv = _pre_call(x, W_enc, b_enc, conv_W[0], degp)
    acc0 = _scatter_kernel(g0, src2, dst2, zrow)
    g1 = _post_call(acc0, g0, dinv, conv_b[0], bn_g[0], bn_b[0], conv_W[1])
    acc1 = _scatter_kernel(g1, src2, dst2, zrow)
    g2 = _post_call(acc1, g1, dinv, conv_b[1], bn_g[1], bn_b[1], conv_W[2])
    acc2 = _scatter_kernel(g2, src2, dst2, zrow)
    return _final_call(acc2, g2, dinv, conv_b[2], bn_g[2], bn_b[2], batch,
                       mlp1_W, mlp1_b, mlp_bn_g, mlp_bn_b, mlp2_W, mlp2_b,
                       bn2_g, bn2_b, out_W, out_b)
